# baseline (speedup 1.0000x reference)
import functools

import jax
import jax.numpy as jnp
from jax import lax
from jax.experimental import pallas as pl
from jax.experimental.pallas import tpu as pltpu
from jax.experimental.pallas import tpu_sc as plsc

VOCAB = 1_000_000
D = 64
SEQ = 1024
BATCH = 1024
FLAT = BATCH * SEQ

NUM_CORES = 2
NUM_SUBCORES = 16
NW = NUM_CORES * NUM_SUBCORES
PER_W = FLAT // NW
C = 256
NCHUNK = PER_W // C
NBUF = 4
NQ = NCHUNK // NBUF

_mesh = plsc.VectorSubcoreMesh(core_axis_name="c", subcore_axis_name="s")


@functools.partial(
    pl.kernel,
    mesh=_mesh,
    compiler_params=pltpu.CompilerParams(use_tc_tiling_on_sc=False),
    out_type=jax.ShapeDtypeStruct((FLAT, D), jnp.float32),
    scratch_types=[
        pltpu.VMEM((NBUF, C), jnp.int32),
        pltpu.VMEM((NBUF, C, D), jnp.float32),
        pltpu.SemaphoreType.DMA((NBUF,)),
        pltpu.SemaphoreType.DMA((NBUF,)),
        pltpu.SemaphoreType.DMA((NBUF,)),
        pltpu.SemaphoreType.DMA((NBUF,)),
    ],
)
def _embed(ids_hbm, tok_hbm, pos_hbm, out_hbm, idx_v, rows_v,
           gsem, osem, isem, psem):
    wid = lax.axis_index("s") * NUM_CORES + lax.axis_index("c")
    base = wid * PER_W

    def prefill(buf):
        pltpu.async_copy(pos_hbm.at[pl.ds(buf * C, C)], rows_v.at[buf],
                         psem.at[buf])

    def wait_prefill(buf):
        pltpu.make_async_copy(pos_hbm.at[pl.ds(buf * C, C)], rows_v.at[buf],
                              psem.at[buf]).wait()

    def wait_store(buf):
        pltpu.make_async_copy(
            rows_v.at[buf], out_hbm.at[pl.ds(base, C)], osem.at[buf]).wait()

    def wait_gather(buf):
        pltpu.make_async_copy(
            tok_hbm.at[idx_v.at[buf]], rows_v.at[buf], gsem.at[buf]).wait()

    for k in range(NBUF):
        pltpu.async_copy(ids_hbm.at[pl.ds(base + k * C, C)], idx_v.at[k],
                         isem.at[k])
    prefill(0)

    def quad_body(o, carry):
        for b in range(NBUF):
            g = o * NBUF + b
            bn = (b + 1) % NBUF
            bl = (b + NBUF - 2) % NBUF

            @pl.when((o < NQ - 1) | (b < NBUF - 1))
            def _stage_next():
                @pl.when((o > 0) | (b == NBUF - 1))
                def _wait_prev_store():
                    wait_store(bn)

                prefill(bn)

            pltpu.make_async_copy(
                ids_hbm.at[pl.ds(base, C)], idx_v.at[b], isem.at[b]).wait()
            wait_prefill(b)
            pltpu.async_copy(tok_hbm.at[idx_v.at[b]], rows_v.at[b],
                             gsem.at[b], add=True)

            @pl.when((o > 0) | (b >= 2))
            def _retire():
                wait_gather(bl)
                pltpu.async_copy(
                    rows_v.at[bl],
                    out_hbm.at[pl.ds(base + (g - 2) * C, C)], osem.at[bl])

                @pl.when(g + 2 < NCHUNK)
                def _prefetch_idx():
                    pltpu.async_copy(
                        ids_hbm.at[pl.ds(base + (g + 2) * C, C)],
                        idx_v.at[bl], isem.at[bl])
        return carry

    lax.fori_loop(0, NQ, quad_body, 0)

    for g in (NCHUNK - 2, NCHUNK - 1):
        b = g % NBUF
        wait_gather(b)
        pltpu.async_copy(rows_v.at[b], out_hbm.at[pl.ds(base + g * C, C)],
                         osem.at[b])
    for b in range(NBUF):
        wait_store(b)


def kernel(input_ids, token_table, pos_table):
    b, s = input_ids.shape
    ids_flat = input_ids.reshape(FLAT).astype(jnp.int32)
    out = _embed(ids_flat, token_table, pos_table)
    return out.reshape(b, s, D)


# speedup vs baseline: 1.1908x; 1.1908x over previous
import functools

import jax
import jax.numpy as jnp
from jax import lax
from jax.experimental import pallas as pl
from jax.experimental.pallas import tpu as pltpu
from jax.experimental.pallas import tpu_sc as plsc

VOCAB = 1_000_000
D = 64
SEQ = 1024
BATCH = 1024

NUM_CORES = 2
NUM_SUBCORES = 16
NW = NUM_CORES * NUM_SUBCORES
ROWS_W = BATCH // NW
C = 128
CPR = SEQ // C
NCHUNK = ROWS_W * CPR
NBUF = 4
NQ = NCHUNK // NBUF
LANES = 16

_mesh = plsc.VectorSubcoreMesh(core_axis_name="c", subcore_axis_name="s")


@functools.partial(
    pl.kernel,
    mesh=_mesh,
    compiler_params=pltpu.CompilerParams(use_tc_tiling_on_sc=False),
    out_type=jax.ShapeDtypeStruct((BATCH, SEQ, D), jnp.float32),
    scratch_types=[
        pltpu.VMEM((SEQ, D), jnp.float32),
        pltpu.VMEM((NBUF, C), jnp.int32),
        pltpu.VMEM((NBUF, C, D), jnp.float32),
        pltpu.SemaphoreType.DMA((NBUF,)),
        pltpu.SemaphoreType.DMA((NBUF,)),
        pltpu.SemaphoreType.DMA((NBUF,)),
    ],
)
def _embed(ids_hbm, tok_hbm, pos_hbm, out_hbm, pos_v, idx_v, rows_v,
           gsem, osem, isem):
    wid = lax.axis_index("s") * NUM_CORES + lax.axis_index("c")
    row0 = wid * ROWS_W
    pltpu.sync_copy(pos_hbm, pos_v)

    def chunk_row_col(g):
        return row0 + g // CPR, (g % CPR) * C

    def prefill(buf, p0):
        @plsc.parallel_loop(0, C, unroll=4)
        def _fill(i):
            for j in range(D // LANES):
                sl = pl.ds(j * LANES, LANES)
                rows_v[buf, i, sl] = pos_v[p0 + i, sl]

    def load_ids(g, buf):
        r, c0 = chunk_row_col(g)
        pltpu.async_copy(ids_hbm.at[r, pl.ds(c0, C)], idx_v.at[buf],
                         isem.at[buf])

    def store_out(g, buf):
        r, c0 = chunk_row_col(g)
        pltpu.async_copy(rows_v.at[buf], out_hbm.at[r, pl.ds(c0, C), :],
                         osem.at[buf])

    def wait_store(buf):
        pltpu.make_async_copy(rows_v.at[buf], out_hbm.at[0, pl.ds(0, C), :],
                              osem.at[buf]).wait()

    def wait_ids(buf):
        pltpu.make_async_copy(ids_hbm.at[0, pl.ds(0, C)], idx_v.at[buf],
                              isem.at[buf]).wait()

    def wait_gather(buf):
        pltpu.make_async_copy(tok_hbm.at[idx_v.at[buf]], rows_v.at[buf],
                              gsem.at[buf]).wait()

    for k in range(NBUF):
        load_ids(k, k)

    def quad_body(o, carry):
        for b in range(NBUF):
            g = o * NBUF + b
            bl = (b + NBUF - 2) % NBUF

            @pl.when(o > 0)
            def _wait_prev_store():
                wait_store(b)

            prefill(b, (g % CPR) * C)
            wait_ids(b)
            pltpu.async_copy(tok_hbm.at[idx_v.at[b]], rows_v.at[b],
                             gsem.at[b], add=True)

            @pl.when((o > 0) | (b >= 2))
            def _retire():
                wait_gather(bl)
                store_out(g - 2, bl)

                @pl.when(g + 2 < NCHUNK)
                def _prefetch_idx():
                    load_ids(g + 2, bl)
        return carry

    lax.fori_loop(0, NQ, quad_body, 0)

    for g in (NCHUNK - 2, NCHUNK - 1):
        b = g % NBUF
        wait_gather(b)
        store_out(g, b)
    for b in range(NBUF):
        wait_store(b)


def kernel(input_ids, token_table, pos_table):
    return _embed(input_ids.astype(jnp.int32), token_table, pos_table)


# speedup vs baseline: 1.1948x; 1.0033x over previous
import functools

import jax
import jax.numpy as jnp
from jax import lax
from jax.experimental import pallas as pl
from jax.experimental.pallas import tpu as pltpu
from jax.experimental.pallas import tpu_sc as plsc

VOCAB = 1_000_000
D = 64
SEQ = 1024
BATCH = 1024

NUM_CORES = 2
NUM_SUBCORES = 16
NW = NUM_CORES * NUM_SUBCORES
ROWS_W = BATCH // NW
C = 128
CPR = SEQ // C
NCHUNK = ROWS_W * CPR
NBUF = 4
NQ = NCHUNK // NBUF
LANES = 16

_mesh = plsc.VectorSubcoreMesh(core_axis_name="c", subcore_axis_name="s")


@functools.partial(
    pl.kernel,
    mesh=_mesh,
    compiler_params=pltpu.CompilerParams(use_tc_tiling_on_sc=False),
    out_type=jax.ShapeDtypeStruct((BATCH, SEQ, D), jnp.float32),
    scratch_types=[
        pltpu.VMEM((SEQ, D), jnp.float32),
        pltpu.VMEM((NBUF, C), jnp.int32),
        pltpu.VMEM((NBUF, C, D), jnp.float32),
        pltpu.SemaphoreType.DMA((NBUF,)),
        pltpu.SemaphoreType.DMA((NBUF,)),
        pltpu.SemaphoreType.DMA((NBUF,)),
    ],
)
def _embed(ids_hbm, tok_hbm, pos_hbm, out_hbm, pos_v, idx_v, rows_v,
           gsem, osem, isem):
    wid = lax.axis_index("s") * NUM_CORES + lax.axis_index("c")
    row0 = wid * ROWS_W
    pltpu.sync_copy(pos_hbm, pos_v)

    def chunk_row_col(g):
        return row0 + g // CPR, (g % CPR) * C

    def prefill(buf, p0):
        @plsc.parallel_loop(0, C, unroll=4)
        def _fill(i):
            for j in range(D // LANES):
                sl = pl.ds(j * LANES, LANES)
                rows_v[buf, i, sl] = pos_v[p0 + i, sl]

    def load_ids(g, buf):
        pltpu.async_copy(ids_hbm.at[wid * NCHUNK + g], idx_v.at[buf],
                         isem.at[buf])

    def store_out(g, buf):
        r, c0 = chunk_row_col(g)
        pltpu.async_copy(rows_v.at[buf], out_hbm.at[r, pl.ds(c0, C), :],
                         osem.at[buf])

    def wait_store(buf):
        pltpu.make_async_copy(rows_v.at[buf], out_hbm.at[0, pl.ds(0, C), :],
                              osem.at[buf]).wait()

    def wait_ids(buf):
        pltpu.make_async_copy(ids_hbm.at[0], idx_v.at[buf],
                              isem.at[buf]).wait()

    def wait_gather(buf):
        pltpu.make_async_copy(tok_hbm.at[idx_v.at[buf]], rows_v.at[buf],
                              gsem.at[buf]).wait()

    for k in range(NBUF):
        load_ids(k, k)

    def quad_body(o, carry):
        for b in range(NBUF):
            g = o * NBUF + b
            bl = (b + NBUF - 2) % NBUF

            @pl.when(o > 0)
            def _wait_prev_store():
                wait_store(b)

            prefill(b, (g % CPR) * C)
            wait_ids(b)
            pltpu.async_copy(tok_hbm.at[idx_v.at[b]], rows_v.at[b],
                             gsem.at[b], add=True)

            @pl.when((o > 0) | (b >= 2))
            def _retire():
                wait_gather(bl)
                store_out(g - 2, bl)

                @pl.when(g + 2 < NCHUNK)
                def _prefetch_idx():
                    load_ids(g + 2, bl)
        return carry

    lax.fori_loop(0, NQ, quad_body, 0)

    for g in (NCHUNK - 2, NCHUNK - 1):
        b = g % NBUF
        wait_gather(b)
        store_out(g, b)
    for b in range(NBUF):
        wait_store(b)


def kernel(input_ids, token_table, pos_table):
    ids_rows = input_ids.astype(jnp.int32).reshape(BATCH * SEQ // C, C)
    return _embed(ids_rows, token_table, pos_table)
